# Initial kernel scaffold; baseline (speedup 1.0000x reference)
import jax
import jax.numpy as jnp
from jax.experimental import pallas as pl


def kernel(embedding, W1l_nb, W1r_nb, b1_nb, W2l_nb, W2r_nb, b2_nb, Wrec_nb, brec_nb, W1l_co, W1r_co, b1_co, W2l_co, W2r_co, b2_co, Wrec_co, brec_co, edge_index_nb, edge_index_co):
    raise NotImplementedError("write your pallas kernel here")


# speedup vs baseline: 7.7376x; 7.7376x over previous
import functools

import jax
import jax.numpy as jnp
from jax import lax
from jax.experimental import pallas as pl
from jax.experimental.pallas import tpu as pltpu
from jax.experimental.pallas import tpu_sc as plsc

N = 10000
H = 64
E = 320000
NC = 2
NS = 16
NW = NC * NS
C = 80
CH = E // NW // C
NP = 10240
ROWS_PT = NP // NS
WA = 80


def _make_segsum(W):
  mesh = plsc.VectorSubcoreMesh(core_axis_name="c", subcore_axis_name="s",
                                num_cores=NC, num_subcores=NS)

  @functools.partial(
      pl.kernel,
      mesh=mesh,
      out_type=jax.ShapeDtypeStruct((NC, NP, W), jnp.float32),
      compiler_params=pltpu.CompilerParams(use_tc_tiling_on_sc=False),
      scratch_types=[
          pltpu.VMEM_SHARED((NP, W), jnp.float32),
          pltpu.VMEM((CH, C), jnp.int32),
          pltpu.VMEM((CH, C), jnp.int32),
          pltpu.VMEM((C, W), jnp.float32),
          pltpu.VMEM((C, W), jnp.float32),
          pltpu.VMEM((128, W), jnp.float32),
          pltpu.SemaphoreType.DMA,
          pltpu.SemaphoreType.DMA,
      ],
  )
  def segsum(table_hbm, srcr_hbm, dstr_hbm, out_hbm,
             acc, src_v, dst_v, rows0, rows1, zbuf, sem0, sem1):
    c = lax.axis_index("c")
    s = lax.axis_index("s")
    wid = s * NC + c

    zv = jnp.zeros((16,), jnp.float32)

    def zero_body(i, carry):
      for j in range(W // 16):
        zbuf[i, pl.ds(j * 16, 16)] = zv
      return carry

    lax.fori_loop(0, 128, zero_body, 0)
    for k in range(ROWS_PT // 128):
      pltpu.sync_copy(zbuf, acc.at[pl.ds(s * ROWS_PT + k * 128, 128)])
    plsc.subcore_barrier()

    pltpu.sync_copy(srcr_hbm.at[wid], src_v)
    pltpu.sync_copy(dstr_hbm.at[wid], dst_v)

    def gather(i, buf, sem):
      return pltpu.make_async_copy(table_hbm.at[src_v.at[i]], buf, sem)

    def scat(i, buf):
      pltpu.sync_copy(buf, acc.at[dst_v.at[i]], add=True)

    gather(0, rows0, sem0).start()

    def body(k, carry):
      i0 = 2 * k
      gather(i0 + 1, rows1, sem1).start()
      gather(i0, rows0, sem0).wait()
      scat(i0, rows0)
      gather(i0 + 2, rows0, sem0).start()
      gather(i0 + 1, rows1, sem1).wait()
      scat(i0 + 1, rows1)
      return carry

    lax.fori_loop(0, (CH - 1) // 2, body, 0)
    gather(CH - 1, rows0, sem0).wait()
    scat(CH - 1, rows0)

    plsc.subcore_barrier()
    pltpu.sync_copy(acc.at[pl.ds(s * ROWS_PT, ROWS_PT)],
                    out_hbm.at[c].at[pl.ds(s * ROWS_PT, ROWS_PT)])

  return segsum


_segsum_aug = _make_segsum(WA)
_segsum_h = _make_segsum(H)


def _dot_t(a, b):
  return lax.dot_general(a, b, (((1,), (1,)), ((), ())),
                         preferred_element_type=jnp.float32)


BM1 = 1000


def _combine1_body(acc_ref, x_ref, wl_ref, wr_ref, b_ref, o_ref):
  accs = acc_ref[0] + acc_ref[1]
  deg = jnp.maximum(accs[:, H:H + 1], 1.0)
  agg = accs[:, :H] / deg
  h = _dot_t(agg, wl_ref[...]) + b_ref[...] + _dot_t(x_ref[...], wr_ref[...])
  o_ref[...] = jnp.maximum(h, 0.0)


def _combine1(acc1, x, wl, wr, b):
  return pl.pallas_call(
      _combine1_body,
      grid=(N // BM1,),
      in_specs=[
          pl.BlockSpec((NC, BM1, WA), lambda i: (0, i, 0)),
          pl.BlockSpec((BM1, H), lambda i: (i, 0)),
          pl.BlockSpec((H, H), lambda i: (0, 0)),
          pl.BlockSpec((H, H), lambda i: (0, 0)),
          pl.BlockSpec((1, H), lambda i: (0, 0)),
      ],
      out_specs=pl.BlockSpec((BM1, H), lambda i: (i, 0)),
      out_shape=jax.ShapeDtypeStruct((N, H), jnp.float32),
  )(acc1, x, wl, wr, b)


BM2 = 400


def _c2r_body(acc2_ref, acc1_ref, h1_ref, wl_ref, wr_ref, b_ref,
              wrec_ref, brec_ref, h2_ref, rec_ref):
  accs = acc2_ref[0] + acc2_ref[1]
  dacc = acc1_ref[0] + acc1_ref[1]
  deg = jnp.maximum(dacc[:, 0:1], 1.0)
  agg = accs / deg
  h = _dot_t(agg, wl_ref[...]) + b_ref[...] + _dot_t(h1_ref[...], wr_ref[...])
  h2 = jnp.maximum(h, 0.0)
  h2_ref[...] = h2
  rec_ref[...] = _dot_t(h2, wrec_ref[...]) + brec_ref[...]


def _combine2_recon(acc2, acc1d, h1, wl, wr, b, wrec, brec):
  return pl.pallas_call(
      _c2r_body,
      grid=(N // BM2,),
      in_specs=[
          pl.BlockSpec((NC, BM2, H), lambda i: (0, i, 0)),
          pl.BlockSpec((NC, BM2, 16), lambda i: (0, i, 0)),
          pl.BlockSpec((BM2, H), lambda i: (i, 0)),
          pl.BlockSpec((H, H), lambda i: (0, 0)),
          pl.BlockSpec((H, H), lambda i: (0, 0)),
          pl.BlockSpec((1, H), lambda i: (0, 0)),
          pl.BlockSpec((N, H), lambda i: (0, 0)),
          pl.BlockSpec((1, N), lambda i: (0, 0)),
      ],
      out_specs=[
          pl.BlockSpec((BM2, H), lambda i: (i, 0)),
          pl.BlockSpec((BM2, N), lambda i: (i, 0)),
      ],
      out_shape=[
          jax.ShapeDtypeStruct((N, H), jnp.float32),
          jax.ShapeDtypeStruct((N, N), jnp.float32),
      ],
  )(acc2, acc1d, h1, wl, wr, b, wrec, brec)


def kernel(embedding,
           W1l_nb, W1r_nb, b1_nb, W2l_nb, W2r_nb, b2_nb, Wrec_nb, brec_nb,
           W1l_co, W1r_co, b1_co, W2l_co, W2r_co, b2_co, Wrec_co, brec_co,
           edge_index_nb, edge_index_co):
  x = embedding
  xaug = jnp.concatenate(
      [x, jnp.ones((N, 1), jnp.float32), jnp.zeros((N, WA - H - 1), jnp.float32)], axis=1)

  outs = []
  for ei, (W1l, W1r, b1, W2l, W2r, b2, Wrec, brec) in (
      (edge_index_nb, (W1l_nb, W1r_nb, b1_nb, W2l_nb, W2r_nb, b2_nb, Wrec_nb, brec_nb)),
      (edge_index_co, (W1l_co, W1r_co, b1_co, W2l_co, W2r_co, b2_co, Wrec_co, brec_co)),
  ):
    srcr = ei[0].reshape(NW, CH, C)
    dstr = ei[1].reshape(NW, CH, C)
    acc1 = _segsum_aug(xaug, srcr, dstr)
    h1 = _combine1(acc1, x, W1l, W1r, b1.reshape(1, H))
    acc2 = _segsum_h(h1, srcr, dstr)
    h2, rec = _combine2_recon(acc2, acc1[:, :, H:H + 16], h1,
                              W2l, W2r, b2.reshape(1, H), Wrec, brec.reshape(1, N))
    outs += [h2, rec]

  return (x, outs[0], outs[1], outs[2], outs[3])


# speedup vs baseline: 7.7389x; 1.0002x over previous
import functools

import jax
import jax.numpy as jnp
from jax import lax
from jax.experimental import pallas as pl
from jax.experimental.pallas import tpu as pltpu
from jax.experimental.pallas import tpu_sc as plsc

N = 10000
H = 64
E = 320000
NC = 2
NS = 16
NW = NC * NS
C = 80
CH = E // NW // C
NP = 10240
ROWS_PT = NP // NS
WA = 80


def _make_segsum(W):
  mesh = plsc.VectorSubcoreMesh(core_axis_name="c", subcore_axis_name="s",
                                num_cores=NC, num_subcores=NS)

  @functools.partial(
      pl.kernel,
      mesh=mesh,
      out_type=jax.ShapeDtypeStruct((NC, NP, W), jnp.float32),
      compiler_params=pltpu.CompilerParams(use_tc_tiling_on_sc=False),
      scratch_types=[
          pltpu.VMEM_SHARED((NP, W), jnp.float32),
          pltpu.VMEM((CH, C), jnp.int32),
          pltpu.VMEM((CH, C), jnp.int32),
          pltpu.VMEM((C, W), jnp.float32),
          pltpu.VMEM((C, W), jnp.float32),
          pltpu.VMEM((128, W), jnp.float32),
          pltpu.SemaphoreType.DMA,
          pltpu.SemaphoreType.DMA,
      ],
  )
  def segsum(table_hbm, srcr_hbm, dstr_hbm, out_hbm,
             acc, src_v, dst_v, rows0, rows1, zbuf, sem0, sem1):
    c = lax.axis_index("c")
    s = lax.axis_index("s")
    wid = s * NC + c

    zv = jnp.zeros((16,), jnp.float32)

    def zero_body(i, carry):
      for j in range(W // 16):
        zbuf[i, pl.ds(j * 16, 16)] = zv
      return carry

    lax.fori_loop(0, 128, zero_body, 0)
    for k in range(ROWS_PT // 128):
      pltpu.sync_copy(zbuf, acc.at[pl.ds(s * ROWS_PT + k * 128, 128)])
    plsc.subcore_barrier()

    pltpu.sync_copy(srcr_hbm.at[wid], src_v)
    pltpu.sync_copy(dstr_hbm.at[wid], dst_v)

    def gather(i, buf, sem):
      return pltpu.make_async_copy(table_hbm.at[src_v.at[i]], buf, sem)

    def scat(i, buf):
      pltpu.sync_copy(buf, acc.at[dst_v.at[i]], add=True)

    gather(0, rows0, sem0).start()

    def body(k, carry):
      i0 = 2 * k
      gather(i0 + 1, rows1, sem1).start()
      gather(i0, rows0, sem0).wait()
      scat(i0, rows0)
      gather(i0 + 2, rows0, sem0).start()
      gather(i0 + 1, rows1, sem1).wait()
      scat(i0 + 1, rows1)
      return carry

    lax.fori_loop(0, (CH - 1) // 2, body, 0)
    gather(CH - 1, rows0, sem0).wait()
    scat(CH - 1, rows0)

    plsc.subcore_barrier()
    pltpu.sync_copy(acc.at[pl.ds(s * ROWS_PT, ROWS_PT)],
                    out_hbm.at[c].at[pl.ds(s * ROWS_PT, ROWS_PT)])

  return segsum


_segsum_aug = _make_segsum(WA)
_segsum_h = _make_segsum(H)


def _dot_t(a, b):
  return lax.dot_general(a, b, (((1,), (1,)), ((), ())),
                         preferred_element_type=jnp.float32)


BM1 = 1000


def _combine1_body(acc_ref, x_ref, wl_ref, wr_ref, b_ref, o_ref):
  accs = acc_ref[0] + acc_ref[1]
  deg = jnp.maximum(accs[:, H:H + 1], 1.0)
  agg = accs[:, :H] / deg
  h = _dot_t(agg, wl_ref[...]) + b_ref[...] + _dot_t(x_ref[...], wr_ref[...])
  o_ref[...] = jnp.maximum(h, 0.0)


def _combine1(acc1, x, wl, wr, b):
  return pl.pallas_call(
      _combine1_body,
      grid=(N // BM1,),
      in_specs=[
          pl.BlockSpec((NC, BM1, WA), lambda i: (0, i, 0)),
          pl.BlockSpec((BM1, H), lambda i: (i, 0)),
          pl.BlockSpec((H, H), lambda i: (0, 0)),
          pl.BlockSpec((H, H), lambda i: (0, 0)),
          pl.BlockSpec((1, H), lambda i: (0, 0)),
      ],
      out_specs=pl.BlockSpec((BM1, H), lambda i: (i, 0)),
      out_shape=jax.ShapeDtypeStruct((N, H), jnp.float32),
  )(acc1, x, wl, wr, b)


BM2 = 400


def _c2r_body(acc2_ref, acc1_ref, h1_ref, wl_ref, wr_ref, b_ref,
              wrec_ref, brec_ref, h2_ref, rec_ref):
  accs = acc2_ref[0] + acc2_ref[1]
  dacc = acc1_ref[0] + acc1_ref[1]
  deg = jnp.maximum(dacc[:, 0:1], 1.0)
  agg = accs / deg
  h = _dot_t(agg, wl_ref[...]) + b_ref[...] + _dot_t(h1_ref[...], wr_ref[...])
  h2 = jnp.maximum(h, 0.0)
  h2_ref[...] = h2
  rec_ref[...] = _dot_t(h2, wrec_ref[...]) + brec_ref[...]


def _combine2_recon(acc2, acc1d, h1, wl, wr, b, wrec, brec):
  return pl.pallas_call(
      _c2r_body,
      grid=(N // BM2,),
      in_specs=[
          pl.BlockSpec((NC, BM2, H), lambda i: (0, i, 0)),
          pl.BlockSpec((NC, BM2, 16), lambda i: (0, i, 0)),
          pl.BlockSpec((BM2, H), lambda i: (i, 0)),
          pl.BlockSpec((H, H), lambda i: (0, 0)),
          pl.BlockSpec((H, H), lambda i: (0, 0)),
          pl.BlockSpec((1, H), lambda i: (0, 0)),
          pl.BlockSpec((N, H), lambda i: (0, 0)),
          pl.BlockSpec((1, N), lambda i: (0, 0)),
      ],
      out_specs=[
          pl.BlockSpec((BM2, H), lambda i: (i, 0)),
          pl.BlockSpec((BM2, N), lambda i: (i, 0)),
      ],
      out_shape=[
          jax.ShapeDtypeStruct((N, H), jnp.float32),
          jax.ShapeDtypeStruct((N, N), jnp.float32),
      ],
  )(acc2, acc1d, h1, wl, wr, b, wrec, brec)


def kernel(embedding,
           W1l_nb, W1r_nb, b1_nb, W2l_nb, W2r_nb, b2_nb, Wrec_nb, brec_nb,
           W1l_co, W1r_co, b1_co, W2l_co, W2r_co, b2_co, Wrec_co, brec_co,
           edge_index_nb, edge_index_co):
  x = embedding
  xaug = jnp.concatenate(
      [x, jnp.ones((N, 1), jnp.float32), jnp.zeros((N, WA - H - 1), jnp.float32)], axis=1)

  srcr_nb = edge_index_nb[0].reshape(NW, CH, C)
  dstr_nb = edge_index_nb[1].reshape(NW, CH, C)
  srcr_co = edge_index_co[0].reshape(NW, CH, C)
  dstr_co = edge_index_co[1].reshape(NW, CH, C)

  acc1_nb = _segsum_aug(xaug, srcr_nb, dstr_nb)
  acc1_co = _segsum_aug(xaug, srcr_co, dstr_co)
  h1_nb = _combine1(acc1_nb, x, W1l_nb, W1r_nb, b1_nb.reshape(1, H))
  acc2_nb = _segsum_h(h1_nb, srcr_nb, dstr_nb)
  h1_co = _combine1(acc1_co, x, W1l_co, W1r_co, b1_co.reshape(1, H))
  acc2_co = _segsum_h(h1_co, srcr_co, dstr_co)
  h2_nb, rec_nb = _combine2_recon(acc2_nb, acc1_nb[:, :, H:H + 16], h1_nb,
                                  W2l_nb, W2r_nb, b2_nb.reshape(1, H),
                                  Wrec_nb, brec_nb.reshape(1, N))
  h2_co, rec_co = _combine2_recon(acc2_co, acc1_co[:, :, H:H + 16], h1_co,
                                  W2l_co, W2r_co, b2_co.reshape(1, H),
                                  Wrec_co, brec_co.reshape(1, N))

  return (x, h2_nb, rec_nb, h2_co, rec_co)


# speedup vs baseline: 8.7755x; 1.1340x over previous
import functools

import jax
import jax.numpy as jnp
from jax import lax
from jax.experimental import pallas as pl
from jax.experimental.pallas import tpu as pltpu
from jax.experimental.pallas import tpu_sc as plsc

N = 10000
H = 64
E = 320000
NC = 2
NS = 16
NW = NC * NS
C = 80
CH = E // NW // C
NP = 10240
ROWS_PT = NP // NS
WA = 80
NBUF = 8


def _make_segsum(W):
  mesh = plsc.VectorSubcoreMesh(core_axis_name="c", subcore_axis_name="s",
                                num_cores=NC, num_subcores=NS)

  @functools.partial(
      pl.kernel,
      mesh=mesh,
      out_type=jax.ShapeDtypeStruct((NC, NP, W), jnp.float32),
      compiler_params=pltpu.CompilerParams(use_tc_tiling_on_sc=False),
      scratch_types=[
          pltpu.VMEM_SHARED((NP, W), jnp.float32),
          pltpu.VMEM((CH, C), jnp.int32),
          pltpu.VMEM((CH, C), jnp.int32),
          [pltpu.VMEM((C, W), jnp.float32) for _ in range(NBUF)],
          pltpu.VMEM((64, W), jnp.float32),
          [pltpu.SemaphoreType.DMA for _ in range(NBUF)],
          [pltpu.SemaphoreType.DMA for _ in range(NBUF)],
      ],
  )
  def segsum(table_hbm, srcr_hbm, dstr_hbm, out_hbm,
             acc, src_v, dst_v, bufs, zbuf, gsems, ssems):
    c = lax.axis_index("c")
    s = lax.axis_index("s")
    wid = s * NC + c

    zv = jnp.zeros((16,), jnp.float32)

    def zero_body(i, carry):
      for j in range(W // 16):
        zbuf[i, pl.ds(j * 16, 16)] = zv
      return carry

    lax.fori_loop(0, 64, zero_body, 0)
    for k in range(ROWS_PT // 64):
      pltpu.sync_copy(zbuf, acc.at[pl.ds(s * ROWS_PT + k * 64, 64)])
    plsc.subcore_barrier()

    pltpu.sync_copy(srcr_hbm.at[wid], src_v)
    pltpu.sync_copy(dstr_hbm.at[wid], dst_v)

    def gstart(i, j):
      pltpu.make_async_copy(table_hbm.at[src_v.at[i]], bufs[j], gsems[j]).start()

    def gwait(i, j):
      pltpu.make_async_copy(table_hbm.at[src_v.at[i]], bufs[j], gsems[j]).wait()

    def sstart(i, j):
      pltpu.async_copy(bufs[j], acc.at[dst_v.at[i]], ssems[j], add=True)

    def swait(i, j):
      pltpu.make_async_copy(bufs[j], acc.at[dst_v.at[i]], ssems[j]).wait()

    for j in range(NBUF):
      gstart(j, j)

    def round_body(k, carry):
      base = k * NBUF
      for j in range(NBUF):
        gwait(base + j, j)
        sstart(base + j, j)
      for j in range(NBUF):
        swait(base + j, j)

        @pl.when(base + NBUF + j < CH)
        def _():
          gstart(base + NBUF + j, j)
      return carry

    nfull = CH // NBUF
    lax.fori_loop(0, nfull, round_body, 0)
    for j in range(CH - nfull * NBUF):
      gwait(nfull * NBUF + j, j)
      sstart(nfull * NBUF + j, j)
    for j in range(CH - nfull * NBUF):
      swait(nfull * NBUF + j, j)

    plsc.subcore_barrier()
    pltpu.sync_copy(acc.at[pl.ds(s * ROWS_PT, ROWS_PT)],
                    out_hbm.at[c].at[pl.ds(s * ROWS_PT, ROWS_PT)])

  return segsum


_segsum_aug = _make_segsum(WA)
_segsum_h = _make_segsum(H)


def _dot_t(a, b):
  return lax.dot_general(a, b, (((1,), (1,)), ((), ())),
                         preferred_element_type=jnp.float32)


BM1 = 1000


def _combine1_body(acc_ref, x_ref, wl_ref, wr_ref, b_ref, o_ref):
  accs = acc_ref[0] + acc_ref[1]
  deg = jnp.maximum(accs[:, H:H + 1], 1.0)
  agg = accs[:, :H] / deg
  h = _dot_t(agg, wl_ref[...]) + b_ref[...] + _dot_t(x_ref[...], wr_ref[...])
  o_ref[...] = jnp.maximum(h, 0.0)


def _combine1(acc1, x, wl, wr, b):
  return pl.pallas_call(
      _combine1_body,
      grid=(N // BM1,),
      in_specs=[
          pl.BlockSpec((NC, BM1, WA), lambda i: (0, i, 0)),
          pl.BlockSpec((BM1, H), lambda i: (i, 0)),
          pl.BlockSpec((H, H), lambda i: (0, 0)),
          pl.BlockSpec((H, H), lambda i: (0, 0)),
          pl.BlockSpec((1, H), lambda i: (0, 0)),
      ],
      out_specs=pl.BlockSpec((BM1, H), lambda i: (i, 0)),
      out_shape=jax.ShapeDtypeStruct((N, H), jnp.float32),
  )(acc1, x, wl, wr, b)


BM2 = 400


def _c2r_body(acc2_ref, acc1_ref, h1_ref, wl_ref, wr_ref, b_ref,
              wrec_ref, brec_ref, h2_ref, rec_ref):
  accs = acc2_ref[0] + acc2_ref[1]
  dacc = acc1_ref[0] + acc1_ref[1]
  deg = jnp.maximum(dacc[:, 0:1], 1.0)
  agg = accs / deg
  h = _dot_t(agg, wl_ref[...]) + b_ref[...] + _dot_t(h1_ref[...], wr_ref[...])
  h2 = jnp.maximum(h, 0.0)
  h2_ref[...] = h2
  rec_ref[...] = _dot_t(h2, wrec_ref[...]) + brec_ref[...]


def _combine2_recon(acc2, acc1d, h1, wl, wr, b, wrec, brec):
  return pl.pallas_call(
      _c2r_body,
      grid=(N // BM2,),
      in_specs=[
          pl.BlockSpec((NC, BM2, H), lambda i: (0, i, 0)),
          pl.BlockSpec((NC, BM2, 16), lambda i: (0, i, 0)),
          pl.BlockSpec((BM2, H), lambda i: (i, 0)),
          pl.BlockSpec((H, H), lambda i: (0, 0)),
          pl.BlockSpec((H, H), lambda i: (0, 0)),
          pl.BlockSpec((1, H), lambda i: (0, 0)),
          pl.BlockSpec((N, H), lambda i: (0, 0)),
          pl.BlockSpec((1, N), lambda i: (0, 0)),
      ],
      out_specs=[
          pl.BlockSpec((BM2, H), lambda i: (i, 0)),
          pl.BlockSpec((BM2, N), lambda i: (i, 0)),
      ],
      out_shape=[
          jax.ShapeDtypeStruct((N, H), jnp.float32),
          jax.ShapeDtypeStruct((N, N), jnp.float32),
      ],
  )(acc2, acc1d, h1, wl, wr, b, wrec, brec)


def kernel(embedding,
           W1l_nb, W1r_nb, b1_nb, W2l_nb, W2r_nb, b2_nb, Wrec_nb, brec_nb,
           W1l_co, W1r_co, b1_co, W2l_co, W2r_co, b2_co, Wrec_co, brec_co,
           edge_index_nb, edge_index_co):
  x = embedding
  xaug = jnp.concatenate(
      [x, jnp.ones((N, 1), jnp.float32), jnp.zeros((N, WA - H - 1), jnp.float32)], axis=1)

  srcr_nb = edge_index_nb[0].reshape(NW, CH, C)
  dstr_nb = edge_index_nb[1].reshape(NW, CH, C)
  srcr_co = edge_index_co[0].reshape(NW, CH, C)
  dstr_co = edge_index_co[1].reshape(NW, CH, C)

  acc1_nb = _segsum_aug(xaug, srcr_nb, dstr_nb)
  acc1_co = _segsum_aug(xaug, srcr_co, dstr_co)
  h1_nb = _combine1(acc1_nb, x, W1l_nb, W1r_nb, b1_nb.reshape(1, H))
  acc2_nb = _segsum_h(h1_nb, srcr_nb, dstr_nb)
  h1_co = _combine1(acc1_co, x, W1l_co, W1r_co, b1_co.reshape(1, H))
  acc2_co = _segsum_h(h1_co, srcr_co, dstr_co)
  h2_nb, rec_nb = _combine2_recon(acc2_nb, acc1_nb[:, :, H:H + 16], h1_nb,
                                  W2l_nb, W2r_nb, b2_nb.reshape(1, H),
                                  Wrec_nb, brec_nb.reshape(1, N))
  h2_co, rec_co = _combine2_recon(acc2_co, acc1_co[:, :, H:H + 16], h1_co,
                                  W2l_co, W2r_co, b2_co.reshape(1, H),
                                  Wrec_co, brec_co.reshape(1, N))

  return (x, h2_nb, rec_nb, h2_co, rec_co)
